# Initial kernel scaffold; baseline (speedup 1.0000x reference)
import jax
import jax.numpy as jnp
from jax.experimental import pallas as pl


def kernel(x, edge_index, batch, task, embedW, embedB, convW1, convB1, convW2, convB2, bnG, bnB, adDownW, adDownB, adUpW, adUpB, alpha, taskEmb, f1W1, f1b1, f1W2, f1b2, f2W1, f2b1, f2W2, f2b2, fc1W, fc1b, fc2W, fc2b, fc3W, fc3b):
    raise NotImplementedError("write your pallas kernel here")


# speedup vs baseline: 11.4691x; 11.4691x over previous
import functools

import jax
import jax.numpy as jnp
from jax import lax
from jax.experimental import pallas as pl
from jax.experimental.pallas import tpu as pltpu
from jax.experimental.pallas import tpu_sc as plsc

N, E, D, H, G, T, TE, R, L = 10000, 640000, 128, 64, 512, 28, 32, 16, 4

NC, NS = 2, 16
NW = NC * NS
CH = 128
KCH = 157
EPAD = NW * KCH * CH
NPAD = 10112
RPT = NPAD // NS


def _sc_agg(h, srcp, dstp, zrows):
    mesh = plsc.VectorSubcoreMesh(
        core_axis_name="c", subcore_axis_name="s", num_cores=NC, num_subcores=NS
    )

    @functools.partial(
        pl.kernel,
        out_type=jax.ShapeDtypeStruct((NC, NPAD, H), jnp.float32),
        mesh=mesh,
        scratch_types=[
            pltpu.VMEM((KCH, CH), jnp.int32),
            pltpu.VMEM((KCH, CH), jnp.int32),
            pltpu.VMEM((CH, H), jnp.float32),
            pltpu.VMEM_SHARED((NPAD, H), jnp.float32),
            pltpu.SemaphoreType.DMA,
        ],
        compiler_params=pltpu.CompilerParams(use_tc_tiling_on_sc=False),
    )
    def k(h_hbm, src_hbm, dst_hbm, z_hbm, out_hbm, src_v, dst_v, rows_v, acc, sem):
        c = lax.axis_index("c")
        s = lax.axis_index("s")
        w = c * NS + s
        pltpu.sync_copy(src_hbm.at[w], src_v)
        pltpu.sync_copy(dst_hbm.at[w], dst_v)
        pltpu.sync_copy(z_hbm.at[pl.ds(s * RPT, RPT)], acc.at[pl.ds(s * RPT, RPT)])
        plsc.subcore_barrier()

        def body(j, carry):
            pltpu.async_copy(h_hbm.at[src_v.at[j]], rows_v, sem).wait()
            pltpu.sync_copy(rows_v, acc.at[dst_v.at[j]], add=True)
            return carry

        lax.fori_loop(0, KCH, body, 0)
        plsc.subcore_barrier()
        pltpu.sync_copy(
            acc.at[pl.ds(s * RPT, RPT)], out_hbm.at[c, pl.ds(s * RPT, RPT)]
        )

    return k(h, srcp, dstp, zrows)


def _tc_embed(x, w, b):
    def body(x_ref, w_ref, b_ref, o_ref):
        o_ref[...] = (
            jnp.dot(x_ref[...], w_ref[...], preferred_element_type=jnp.float32)
            + b_ref[...]
        )

    return pl.pallas_call(
        body, out_shape=jax.ShapeDtypeStruct((N, H), jnp.float32)
    )(x, w, b)


def _tc_mlp(h, agg2, w1, b1, w2, b2, gam, bet):
    def body(h_ref, a_ref, w1_ref, b1_ref, w2_ref, b2_ref, g_ref, be_ref, o_ref):
        m = h_ref[...] + a_ref[0, :N, :] + a_ref[1, :N, :]
        m = jax.nn.relu(
            jnp.dot(m, w1_ref[...], preferred_element_type=jnp.float32) + b1_ref[...]
        )
        m = jnp.dot(m, w2_ref[...], preferred_element_type=jnp.float32) + b2_ref[...]
        mu = jnp.mean(m, axis=0, keepdims=True)
        var = jnp.mean((m - mu) ** 2, axis=0, keepdims=True)
        m = (m - mu) * jax.lax.rsqrt(var + 1e-5) * g_ref[...] + be_ref[...]
        o_ref[...] = jax.nn.relu(m)

    return pl.pallas_call(
        body, out_shape=jax.ShapeDtypeStruct((N, H), jnp.float32)
    )(h, agg2, w1, b1, w2, b2, gam, bet)


def _tc_head(h, batch_row, task_col, adDownW, adDownB, adUpW, adUpB, alpha_row,
             taskEmb, f1W1, f1b1, f1W2, f1b2, f2W1, f2b1, f2W2, f2b2,
             fc1W, fc1b, fc2W, fc2b, fc3W, fc3b):
    NCHUNK = 8
    CSZ = N // NCHUNK

    def body(h_ref, b_ref, t_ref, dW_ref, dB_ref, uW_ref, uB_ref, al_ref,
             te_ref, f1W1_ref, f1b1_ref, f1W2_ref, f1b2_ref,
             f2W1_ref, f2b1_ref, f2W2_ref, f2b2_ref,
             fc1W_ref, fc1b_ref, fc2W_ref, fc2b_ref, fc3W_ref, fc3b_ref, o_ref):
        iota_g = lax.broadcasted_iota(jnp.int32, (G, 1), 0)
        pool = jnp.zeros((G, H), jnp.float32)
        for ci in range(NCHUNK):
            bchunk = b_ref[0:1, ci * CSZ:(ci + 1) * CSZ]
            onehot = (iota_g == bchunk).astype(jnp.float32)
            pool = pool + jnp.dot(
                onehot, h_ref[ci * CSZ:(ci + 1) * CSZ, :],
                preferred_element_type=jnp.float32,
            )

        task = t_ref[...]
        iota_t = lax.broadcasted_iota(jnp.int32, (1, T), 1)
        oh_gt = (task == iota_t).astype(jnp.float32)
        iota_t2 = lax.broadcasted_iota(jnp.int32, (T, 1), 0)
        oh_tg = (iota_t2 == jnp.reshape(task, (1, G))).astype(jnp.float32)

        down_b = jnp.dot(oh_gt, dB_ref[...], preferred_element_type=jnp.float32)
        up_b = jnp.dot(oh_gt, uB_ref[...], preferred_element_type=jnp.float32)
        mid = jnp.zeros((G, R), jnp.float32)
        for t in range(T):
            sel = oh_gt[:, t:t + 1]
            mid = mid + sel * jnp.dot(
                pool, dW_ref[t], preferred_element_type=jnp.float32
            )
        mid = jax.nn.relu(mid + down_b)
        g2 = jnp.zeros((G, H), jnp.float32)
        for t in range(T):
            sel = oh_gt[:, t:t + 1]
            g2 = g2 + sel * jnp.dot(
                mid, uW_ref[t], preferred_element_type=jnp.float32
            )
        g2 = g2 + up_b + pool

        counts = jnp.sum(oh_tg, axis=1, keepdims=True)
        sums = jnp.dot(oh_tg, g2, preferred_element_type=jnp.float32)
        means = sums / jnp.maximum(counts, 1.0)
        present = (counts > 0).astype(jnp.float32)
        fused = jnp.dot(
            al_ref[...], means * present, preferred_element_type=jnp.float32
        )
        g2 = g2 + oh_gt[:, 0:1] * fused

        te = jnp.dot(oh_gt, te_ref[...], preferred_element_type=jnp.float32)

        def film(W1, b1, W2, b2):
            o = jax.nn.relu(
                jnp.dot(te, W1, preferred_element_type=jnp.float32) + b1
            )
            o = jnp.dot(o, W2, preferred_element_type=jnp.float32) + b2
            gr = o[:, :H]
            br = o[:, H:]
            return 1.0 + 0.5 * jnp.tanh(gr), jnp.tanh(br)

        gm1, bt1 = film(f1W1_ref[...], f1b1_ref[...], f1W2_ref[...], f1b2_ref[...])
        z = gm1 * g2 + bt1
        z = jax.nn.relu(
            jnp.dot(z, fc1W_ref[...], preferred_element_type=jnp.float32)
            + fc1b_ref[...]
        )
        gm2, bt2 = film(f2W1_ref[...], f2b1_ref[...], f2W2_ref[...], f2b2_ref[...])
        z = gm2 * z + bt2
        z = jax.nn.relu(
            jnp.dot(z, fc2W_ref[...], preferred_element_type=jnp.float32)
            + fc2b_ref[...]
        )
        o_ref[...] = (
            jnp.dot(z, fc3W_ref[...], preferred_element_type=jnp.float32)
            + fc3b_ref[...]
        )

    return pl.pallas_call(
        body, out_shape=jax.ShapeDtypeStruct((G, 1), jnp.float32)
    )(h, batch_row, task_col, adDownW, adDownB, adUpW, adUpB, alpha_row,
      taskEmb, f1W1, f1b1, f1W2, f1b2, f2W1, f2b1, f2W2, f2b2,
      fc1W, fc1b, fc2W, fc2b, fc3W, fc3b)


def kernel(x, edge_index, batch, task, embedW, embedB, convW1, convB1, convW2,
           convB2, bnG, bnB, adDownW, adDownB, adUpW, adUpB, alpha, taskEmb,
           f1W1, f1b1, f1W2, f1b2, f2W1, f2b1, f2W2, f2b2, fc1W, fc1b, fc2W,
           fc2b, fc3W, fc3b):
    src, dst = edge_index[0], edge_index[1]
    pad = EPAD - E
    srcp = jnp.concatenate([src, jnp.zeros((pad,), jnp.int32)]).reshape(NW, KCH, CH)
    dstp = jnp.concatenate([dst, jnp.full((pad,), N, jnp.int32)]).reshape(NW, KCH, CH)
    zrows = jnp.zeros((NPAD, H), jnp.float32)

    h = _tc_embed(x, embedW, embedB.reshape(1, H))
    for l in range(L):
        agg2 = _sc_agg(h, srcp, dstp, zrows)
        h = _tc_mlp(
            h, agg2,
            convW1[l], convB1[l].reshape(1, H),
            convW2[l], convB2[l].reshape(1, H),
            bnG[l].reshape(1, H), bnB[l].reshape(1, H),
        )

    out = _tc_head(
        h,
        batch.reshape(1, N),
        task.reshape(G, 1),
        adDownW, adDownB, adUpW, adUpB,
        alpha[0:1, :],
        taskEmb, f1W1, f1b1.reshape(1, TE), f1W2, f1b2.reshape(1, 2 * H),
        f2W1, f2b1.reshape(1, TE), f2W2, f2b2.reshape(1, 2 * H),
        fc1W, fc1b.reshape(1, H), fc2W, fc2b.reshape(1, H),
        fc3W, fc3b.reshape(1, 1),
    )
    return out.reshape(-1)
